# baseline (speedup 1.0000x reference)
import functools

import jax
import jax.numpy as jnp
from jax import lax
from jax.experimental import pallas as pl
from jax.experimental.pallas import tpu as pltpu
from jax.experimental.pallas import tpu_sc as plsc

ROWS = 128
COLS = 32768
LANES = 16
NUM_CORES = 2
NUM_SUBCORES = 16
NUM_WORKERS = NUM_CORES * NUM_SUBCORES
ROWS_PER_WORKER = ROWS // NUM_WORKERS
VECS = COLS // LANES
UNROLL = 8


def _row_argmin(buf, lane_iota):
    def body(i, carry):
        minv, mini = carry
        base = i * (LANES * UNROLL)
        for u in range(UNROLL):
            off = base + u * LANES
            v = buf[pl.ds(off, LANES)]
            idxv = lane_iota + off
            pred = v < minv
            minv = jnp.where(pred, v, minv)
            mini = jnp.where(pred, idxv, mini)
        return minv, mini

    minv0 = jnp.full((LANES,), jnp.inf, jnp.float32)
    mini0 = jnp.zeros((LANES,), jnp.int32)
    minv, mini = lax.fori_loop(0, VECS // UNROLL, body, (minv0, mini0))
    m = jnp.min(minv)
    cand = jnp.where(minv == m, mini, jnp.int32(COLS))
    return jnp.min(cand)


@functools.partial(
    pl.kernel,
    out_type=jax.ShapeDtypeStruct((NUM_WORKERS, LANES), jnp.int32),
    mesh=plsc.VectorSubcoreMesh(
        core_axis_name="c",
        subcore_axis_name="s",
        num_cores=NUM_CORES,
        num_subcores=NUM_SUBCORES,
    ),
    scratch_types=[
        pltpu.VMEM((COLS,), jnp.float32),
        pltpu.VMEM((COLS,), jnp.float32),
        pltpu.VMEM((LANES,), jnp.int32),
        pltpu.SemaphoreType.DMA,
        pltpu.SemaphoreType.DMA,
    ],
    compiler_params=pltpu.CompilerParams(needs_layout_passes=False),
)
def _argmin_sc(x_hbm, out_hbm, buf0, buf1, res_ref, sem0, sem1):
    wid = lax.axis_index("s") * NUM_CORES + lax.axis_index("c")
    base = wid * ROWS_PER_WORKER
    lane_iota = lax.iota(jnp.int32, LANES)
    bufs = (buf0, buf1)
    sems = (sem0, sem1)

    copies = [None] * ROWS_PER_WORKER
    copies[0] = pltpu.async_copy(x_hbm.at[base], buf0, sem0)
    res = jnp.zeros((LANES,), jnp.int32)
    for j in range(ROWS_PER_WORKER):
        copies[j].wait()
        if j + 1 < ROWS_PER_WORKER:
            copies[j + 1] = pltpu.async_copy(
                x_hbm.at[base + j + 1], bufs[(j + 1) % 2], sems[(j + 1) % 2]
            )
        s = _row_argmin(bufs[j % 2], lane_iota)
        res = jnp.where(lane_iota == j, s, res)
    res_ref[...] = res
    pltpu.sync_copy(res_ref, out_hbm.at[wid])


def kernel(x):
    padded = _argmin_sc(x)
    flat = padded[:, :ROWS_PER_WORKER].reshape(ROWS)
    return (flat.reshape(ROWS, 1), flat)


# speedup vs baseline: 1.0464x; 1.0464x over previous
import functools

import jax
import jax.numpy as jnp
from jax import lax
from jax.experimental import pallas as pl
from jax.experimental.pallas import tpu as pltpu
from jax.experimental.pallas import tpu_sc as plsc

ROWS = 128
COLS = 32768
LANES = 16
NUM_CORES = 2
NUM_SUBCORES = 16
NUM_WORKERS = NUM_CORES * NUM_SUBCORES
ROWS_PER_WORKER = ROWS // NUM_WORKERS
VECS = COLS // LANES
UNROLL = 16
NACC = 4


def _row_argmin(buf, lane_iota):
    def body(i, carry):
        minvs, minis = carry
        minvs = list(minvs)
        minis = list(minis)
        base = i * (LANES * UNROLL)
        for u in range(UNROLL):
            k = u % NACC
            off = base + u * LANES
            v = buf[pl.ds(off, LANES)]
            idxv = lane_iota + off
            pred = v < minvs[k]
            minvs[k] = jnp.where(pred, v, minvs[k])
            minis[k] = jnp.where(pred, idxv, minis[k])
        return tuple(minvs), tuple(minis)

    minv0 = jnp.full((LANES,), jnp.inf, jnp.float32)
    mini0 = jnp.zeros((LANES,), jnp.int32)
    minvs, minis = lax.fori_loop(
        0, VECS // UNROLL, body, ((minv0,) * NACC, (mini0,) * NACC)
    )
    minv, mini = minvs[0], minis[0]
    for k in range(1, NACC):
        pred = (minvs[k] < minv) | ((minvs[k] == minv) & (minis[k] < mini))
        minv = jnp.where(pred, minvs[k], minv)
        mini = jnp.where(pred, minis[k], mini)
    m = jnp.min(minv)
    cand = jnp.where(minv == m, mini, jnp.int32(COLS))
    return jnp.min(cand)


@functools.partial(
    pl.kernel,
    out_type=jax.ShapeDtypeStruct((NUM_WORKERS, LANES), jnp.int32),
    mesh=plsc.VectorSubcoreMesh(
        core_axis_name="c",
        subcore_axis_name="s",
        num_cores=NUM_CORES,
        num_subcores=NUM_SUBCORES,
    ),
    scratch_types=[
        pltpu.VMEM((COLS,), jnp.float32),
        pltpu.VMEM((COLS,), jnp.float32),
        pltpu.VMEM((LANES,), jnp.int32),
        pltpu.SemaphoreType.DMA,
        pltpu.SemaphoreType.DMA,
    ],
    compiler_params=pltpu.CompilerParams(needs_layout_passes=False),
)
def _argmin_sc(x_hbm, out_hbm, buf0, buf1, res_ref, sem0, sem1):
    wid = lax.axis_index("s") * NUM_CORES + lax.axis_index("c")
    base = wid * ROWS_PER_WORKER
    lane_iota = lax.iota(jnp.int32, LANES)
    bufs = (buf0, buf1)
    sems = (sem0, sem1)

    copies = [None] * ROWS_PER_WORKER
    copies[0] = pltpu.async_copy(x_hbm.at[base], buf0, sem0)
    res = jnp.zeros((LANES,), jnp.int32)
    for j in range(ROWS_PER_WORKER):
        copies[j].wait()
        if j + 1 < ROWS_PER_WORKER:
            copies[j + 1] = pltpu.async_copy(
                x_hbm.at[base + j + 1], bufs[(j + 1) % 2], sems[(j + 1) % 2]
            )
        s = _row_argmin(bufs[j % 2], lane_iota)
        res = jnp.where(lane_iota == j, s, res)
    res_ref[...] = res
    pltpu.sync_copy(res_ref, out_hbm.at[wid])


def kernel(x):
    padded = _argmin_sc(x)
    flat = padded[:, :ROWS_PER_WORKER].reshape(ROWS)
    return (flat.reshape(ROWS, 1), flat)
